# baseline (speedup 1.0000x reference)
import functools

import jax
import jax.numpy as jnp
from jax import lax
from jax.experimental import pallas as pl
from jax.experimental.pallas import tpu as pltpu, tpu_sc as plsc

_N = 10000
_E = 320000
_F = 128
_C = 64
_CH = 128
_EP = 327680
_NCHUNK = _EP // _CH
_CPT = _NCHUNK // 16
_CPT2 = _NCHUNK // 32
_SENT = 10239
_NP = 10240
_RPT = 640
_R = 400
_NB = 4

_mesh = plsc.VectorSubcoreMesh(
    core_axis_name="c", subcore_axis_name="s", num_cores=2, num_subcores=16)


def _deg_body(src_hbm, dst_hbm, zc_hbm, out_hbm, idx_v, ones_v, stage_v, acc_sh):
    c = lax.axis_index("c")
    s = lax.axis_index("s")
    pltpu.sync_copy(zc_hbm.at[pl.ds(0, _RPT)], stage_v)
    pltpu.sync_copy(stage_v, acc_sh.at[pl.ds(s * _RPT, _RPT)])
    for k in range(_CH // 16):
        ones_v[pl.ds(k * 16, 16)] = jnp.ones((16,), jnp.float32)

    @pl.when(c == 0)
    def _():
        pltpu.sync_copy(src_hbm.at[pl.ds(s * _CPT, _CPT)], idx_v)

    @pl.when(c == 1)
    def _():
        pltpu.sync_copy(dst_hbm.at[pl.ds(s * _CPT, _CPT)], idx_v)

    plsc.subcore_barrier()

    def body(k, carry):
        pltpu.sync_copy(ones_v, acc_sh.at[idx_v.at[k]], add=True)
        return carry

    lax.fori_loop(0, _CPT, body, 0)
    plsc.subcore_barrier()
    pltpu.sync_copy(acc_sh.at[pl.ds(s * _RPT, _RPT)], stage_v)
    pltpu.sync_copy(stage_v, out_hbm.at[c, pl.ds(s * _RPT, _RPT)])


_deg_call = pl.kernel(
    _deg_body,
    out_type=jax.ShapeDtypeStruct((2, _NP), jnp.float32),
    mesh=_mesh,
    compiler_params=pltpu.CompilerParams(use_tc_tiling_on_sc=False),
    scratch_types=[
        pltpu.VMEM((_CPT, _CH), jnp.int32),
        pltpu.VMEM((_CH,), jnp.float32),
        pltpu.VMEM((_RPT,), jnp.float32),
        pltpu.VMEM_SHARED((_NP,), jnp.float32),
    ],
)


def _prop_body(edge_split, ta_hbm, tb_hbm, src_hbm, dst_hbm, zr_hbm, out_hbm,
               sidx, didx, zidx, rows, zbuf, acc_sh, gsems, ssems, zsem):
    c = lax.axis_index("c")
    s = lax.axis_index("s")
    cpt = _CPT2 if edge_split else _CPT
    chunk0 = (c * 16 + s) * cpt if edge_split else s * cpt

    iota = lax.iota(jnp.int32, 16)
    for k in range(_RPT // _CH):
        for v in range(_CH // 16):
            zidx[k, pl.ds(v * 16, 16)] = iota + (s * _RPT + k * _CH + v * 16)

    pltpu.sync_copy(zr_hbm.at[pl.ds(0, _CH)], zbuf)
    for k in range(_RPT // _CH):
        pltpu.async_copy(zbuf, acc_sh.at[zidx.at[k]], zsem)
    for k in range(_RPT // _CH):
        pltpu.make_async_copy(zbuf, acc_sh.at[zidx.at[k]], zsem).wait()

    pltpu.sync_copy(src_hbm.at[pl.ds(chunk0, cpt)], sidx.at[pl.ds(0, cpt)])
    pltpu.sync_copy(dst_hbm.at[pl.ds(chunk0, cpt)], didx.at[pl.ds(0, cpt)])
    plsc.subcore_barrier()

    def run(tbl_hbm):
        def gather(k, b):
            pltpu.async_copy(tbl_hbm.at[sidx.at[k]], rows.at[b], gsems.at[b])

        def wait_gather(k, b):
            pltpu.make_async_copy(tbl_hbm.at[sidx.at[k]], rows.at[b], gsems.at[b]).wait()

        def scatter(k, b):
            pltpu.async_copy(rows.at[b], acc_sh.at[didx.at[k]], ssems.at[b], add=True)

        def wait_scatter(k, b):
            pltpu.make_async_copy(rows.at[b], acc_sh.at[didx.at[k]], ssems.at[b]).wait()

        for b in range(_NB):
            gather(b, b)

        def body(j, carry):
            k = _NB * j
            for b in range(_NB):
                wait_gather(k + b, b)
                scatter(k + b, b)
            for b in range(_NB):
                kn = k + _NB + b

                @pl.when(kn < cpt)
                def _(b=b, kn=kn):
                    wait_scatter(kn - _NB, b)
                    gather(kn, b)
            return carry

        lax.fori_loop(0, cpt // _NB, body, 0)
        for b in range(_NB):
            wait_scatter(cpt - _NB + b, b)

    @pl.when(c == 0)
    def _():
        run(ta_hbm)

    @pl.when(c == 1)
    def _():
        run(tb_hbm)

    plsc.subcore_barrier()
    for k in range(_RPT // _CH):
        pltpu.async_copy(acc_sh.at[zidx.at[k]], rows.at[k % _NB], gsems.at[k % _NB])
        pltpu.make_async_copy(acc_sh.at[zidx.at[k]], rows.at[k % _NB], gsems.at[k % _NB]).wait()
        pltpu.sync_copy(rows.at[k % _NB],
                        out_hbm.at[c, pl.ds(s * _RPT + k * _CH, _CH)])


def _make_prop(edge_split):
    return pl.kernel(
        functools.partial(_prop_body, edge_split),
        out_type=jax.ShapeDtypeStruct((2, _NP, _C), jnp.float32),
        mesh=_mesh,
        compiler_params=pltpu.CompilerParams(use_tc_tiling_on_sc=False),
        scratch_types=[
            pltpu.VMEM((_CPT, _CH), jnp.int32),
            pltpu.VMEM((_CPT, _CH), jnp.int32),
            pltpu.VMEM((_RPT // _CH, _CH), jnp.int32),
            pltpu.VMEM((_NB, _CH, _C), jnp.float32),
            pltpu.VMEM((_CH, _C), jnp.float32),
            pltpu.VMEM_SHARED((_NP, _C), jnp.float32),
            pltpu.SemaphoreType.DMA((_NB,)),
            pltpu.SemaphoreType.DMA((_NB,)),
            pltpu.SemaphoreType.DMA,
        ],
    )


_prop1_call = _make_prop(False)
_prop2_call = _make_prop(True)


def _scale_split_body(x_ref, degt_ref, norms_ref, xa_ref, xb_ref):
    ns = lax.rsqrt(jnp.maximum(degt_ref[:, 0:1], 1.0))
    nd = lax.rsqrt(jnp.maximum(degt_ref[:, 1:2], 1.0))
    xs = x_ref[...] * ns
    xa_ref[...] = xs[:, :_C]
    xb_ref[...] = xs[:, _C:]
    norms_ref[...] = jnp.concatenate([ns, nd], axis=1)


_scale_split_call = pl.pallas_call(
    _scale_split_body,
    grid=(_N // _R,),
    in_specs=[
        pl.BlockSpec((_R, _F), lambda i: (i, 0)),
        pl.BlockSpec((_R, 2), lambda i: (i, 0)),
    ],
    out_specs=[
        pl.BlockSpec((_R, 2), lambda i: (i, 0)),
        pl.BlockSpec((_R, _C), lambda i: (i, 0)),
        pl.BlockSpec((_R, _C), lambda i: (i, 0)),
    ],
    out_shape=[
        jax.ShapeDtypeStruct((_N, 2), jnp.float32),
        jax.ShapeDtypeStruct((_NP, _C), jnp.float32),
        jax.ShapeDtypeStruct((_NP, _C), jnp.float32),
    ],
)


def _mlp_body(s1_ref, norms_ref, w1_ref, b1_ref, w2_ref, t2_ref):
    agg = jnp.concatenate([s1_ref[0], s1_ref[1]], axis=1)
    h = agg * norms_ref[:, 1:2]
    h = jnp.dot(h, w1_ref[...], preferred_element_type=jnp.float32) + b1_ref[...]
    h = jnp.maximum(h, 0.0)
    t2 = jnp.dot(h, w2_ref[...], preferred_element_type=jnp.float32)
    t2_ref[...] = t2 * norms_ref[:, 0:1]


_mlp_call = pl.pallas_call(
    _mlp_body,
    grid=(_N // _R,),
    in_specs=[
        pl.BlockSpec((2, _R, _C), lambda i: (0, i, 0)),
        pl.BlockSpec((_R, 2), lambda i: (i, 0)),
        pl.BlockSpec((_F, _F), lambda i: (0, 0)),
        pl.BlockSpec((1, _F), lambda i: (0, 0)),
        pl.BlockSpec((_F, _C), lambda i: (0, 0)),
    ],
    out_specs=pl.BlockSpec((_R, _C), lambda i: (i, 0)),
    out_shape=jax.ShapeDtypeStruct((_NP, _C), jnp.float32),
)


def _final_body(s2_ref, norms_ref, b2_ref, out_ref):
    agg = s2_ref[0] + s2_ref[1]
    out_ref[...] = agg * norms_ref[:, 1:2] + b2_ref[...]


_final_call = pl.pallas_call(
    _final_body,
    grid=(_N // _R,),
    in_specs=[
        pl.BlockSpec((2, _R, _C), lambda i: (0, i, 0)),
        pl.BlockSpec((_R, 2), lambda i: (i, 0)),
        pl.BlockSpec((1, _C), lambda i: (0, 0)),
    ],
    out_specs=pl.BlockSpec((_R, _C), lambda i: (i, 0)),
    out_shape=jax.ShapeDtypeStruct((_N, _C), jnp.float32),
)


def kernel(x, edge_index, W1, b1, W2, b2):
    pad = jnp.full((_EP - _E,), _SENT, jnp.int32)
    src = jnp.concatenate([edge_index[0].astype(jnp.int32), pad]).reshape(_NCHUNK, _CH)
    dst = jnp.concatenate([edge_index[1].astype(jnp.int32), pad]).reshape(_NCHUNK, _CH)
    zc = jnp.zeros((_RPT,), jnp.float32)
    zr = jnp.zeros((_CH, _C), jnp.float32)

    degs = _deg_call(src, dst, zc)
    norms, xa, xb = _scale_split_call(x, degs[:, :_N].T)
    s1 = _prop1_call(xa, xb, src, dst, zr)
    t2 = _mlp_call(s1, norms, W1, b1.reshape(1, -1), W2)
    s2 = _prop2_call(t2, t2, src, dst, zr)
    return _final_call(s2, norms, b2.reshape(1, -1))


# speedup vs baseline: 2.2273x; 2.2273x over previous
import functools

import jax
import jax.numpy as jnp
from jax import lax
from jax.experimental import pallas as pl
from jax.experimental.pallas import tpu as pltpu, tpu_sc as plsc

_N = 10000
_E = 320000
_F = 128
_C = 64
_CH = 128
_EP = 327680
_NCHUNK = _EP // _CH
_CPT = _NCHUNK // 16
_CPT2 = _NCHUNK // 32
_SENT = 10239
_NP = 10240
_RPT = 640
_R = 400
_NB = 4

_mesh = plsc.VectorSubcoreMesh(
    core_axis_name="c", subcore_axis_name="s", num_cores=2, num_subcores=16)


def _deg_body(src_hbm, dst_hbm, zc_hbm, out_hbm, idx_v, ones_v, stage_v, acc_sh):
    c = lax.axis_index("c")
    s = lax.axis_index("s")
    pltpu.sync_copy(zc_hbm.at[pl.ds(0, _RPT)], stage_v)
    pltpu.sync_copy(stage_v, acc_sh.at[pl.ds(s * _RPT, _RPT)])
    for k in range(_CH // 16):
        ones_v[pl.ds(k * 16, 16)] = jnp.ones((16,), jnp.float32)

    @pl.when(c == 0)
    def _():
        pltpu.sync_copy(src_hbm.at[pl.ds(s * _CPT, _CPT)], idx_v)

    @pl.when(c == 1)
    def _():
        pltpu.sync_copy(dst_hbm.at[pl.ds(s * _CPT, _CPT)], idx_v)

    plsc.subcore_barrier()

    def body(k, carry):
        pltpu.sync_copy(ones_v, acc_sh.at[idx_v.at[k]], add=True)
        return carry

    lax.fori_loop(0, _CPT, body, 0)
    plsc.subcore_barrier()
    pltpu.sync_copy(acc_sh.at[pl.ds(s * _RPT, _RPT)], stage_v)
    pltpu.sync_copy(stage_v, out_hbm.at[c, pl.ds(s * _RPT, _RPT)])


_deg_call = pl.kernel(
    _deg_body,
    out_type=jax.ShapeDtypeStruct((2, _NP), jnp.float32),
    mesh=_mesh,
    compiler_params=pltpu.CompilerParams(use_tc_tiling_on_sc=False),
    scratch_types=[
        pltpu.VMEM((_CPT, _CH), jnp.int32),
        pltpu.VMEM((_CH,), jnp.float32),
        pltpu.VMEM((_RPT,), jnp.float32),
        pltpu.VMEM_SHARED((_NP,), jnp.float32),
    ],
)


def _prop_body(edge_split, ta_hbm, tb_hbm, src_hbm, dst_hbm, zr_hbm, out_hbm,
               sidx, didx, zidx, rows, zbuf, acc_sh, gsems, ssems, zsem):
    c = lax.axis_index("c")
    s = lax.axis_index("s")
    cpt = _CPT2 if edge_split else _CPT
    chunk0 = (c * 16 + s) * cpt if edge_split else s * cpt

    iota = lax.iota(jnp.int32, 16)
    for k in range(_RPT // _CH):
        for v in range(_CH // 16):
            zidx[k, pl.ds(v * 16, 16)] = iota + (s * _RPT + k * _CH + v * 16)

    pltpu.sync_copy(zr_hbm.at[pl.ds(0, _CH)], zbuf)
    for k in range(_RPT // _CH):
        pltpu.async_copy(zbuf, acc_sh.at[zidx.at[k]], zsem)
    for k in range(_RPT // _CH):
        pltpu.make_async_copy(zbuf, acc_sh.at[zidx.at[k]], zsem).wait()

    pltpu.sync_copy(src_hbm.at[pl.ds(chunk0, cpt)], sidx.at[pl.ds(0, cpt)])
    pltpu.sync_copy(dst_hbm.at[pl.ds(chunk0, cpt)], didx.at[pl.ds(0, cpt)])
    plsc.subcore_barrier()

    def run(tbl_hbm):
        def gather(k, b):
            pltpu.async_copy(tbl_hbm.at[sidx.at[k]], rows.at[b], gsems.at[b])

        def wait_gather(k, b):
            pltpu.make_async_copy(tbl_hbm.at[sidx.at[k]], rows.at[b], gsems.at[b]).wait()

        def scatter(k, b):
            pltpu.async_copy(rows.at[b], acc_sh.at[didx.at[k]], ssems.at[b], add=True)

        def wait_scatter(k, b):
            pltpu.make_async_copy(rows.at[b], acc_sh.at[didx.at[k]], ssems.at[b]).wait()

        for b in range(_NB):
            gather(b, b)

        def body(j, carry):
            k = _NB * j
            for b in range(_NB):
                wait_gather(k + b, b)
                scatter(k + b, b)
            for b in range(_NB):
                kn = k + _NB + b

                @pl.when(kn < cpt)
                def _(b=b, kn=kn):
                    wait_scatter(kn - _NB, b)
                    gather(kn, b)
            return carry

        lax.fori_loop(0, cpt // _NB, body, 0)
        for b in range(_NB):
            wait_scatter(cpt - _NB + b, b)

    @pl.when(c == 0)
    def _():
        run(ta_hbm)

    @pl.when(c == 1)
    def _():
        run(tb_hbm)

    plsc.subcore_barrier()
    for k in range(_RPT // _CH):
        pltpu.async_copy(acc_sh.at[zidx.at[k]], rows.at[k % _NB], gsems.at[k % _NB])
        pltpu.make_async_copy(acc_sh.at[zidx.at[k]], rows.at[k % _NB], gsems.at[k % _NB]).wait()
        pltpu.sync_copy(rows.at[k % _NB],
                        out_hbm.at[c, pl.ds(s * _RPT + k * _CH, _CH)])


def _make_prop(edge_split):
    return pl.kernel(
        functools.partial(_prop_body, edge_split),
        out_type=jax.ShapeDtypeStruct((2, _NP, _C), jnp.float32),
        mesh=_mesh,
        compiler_params=pltpu.CompilerParams(use_tc_tiling_on_sc=False),
        scratch_types=[
            pltpu.VMEM((_CPT, _CH), jnp.int32),
            pltpu.VMEM((_CPT, _CH), jnp.int32),
            pltpu.VMEM((_RPT // _CH, _CH), jnp.int32),
            pltpu.VMEM((_NB, _CH, _C), jnp.float32),
            pltpu.VMEM((_CH, _C), jnp.float32),
            pltpu.VMEM_SHARED((_NP, _C), jnp.float32),
            pltpu.SemaphoreType.DMA((_NB,)),
            pltpu.SemaphoreType.DMA((_NB,)),
            pltpu.SemaphoreType.DMA,
        ],
    )


_prop1_call = _make_prop(False)
_prop2_call = _make_prop(True)


def _scale_split_body(x_ref, degt_ref, norms_ref, xa_ref, xb_ref):
    ns = lax.rsqrt(jnp.maximum(degt_ref[:, 0:1], 1.0))
    nd = lax.rsqrt(jnp.maximum(degt_ref[:, 1:2], 1.0))
    xs = x_ref[...] * ns
    xa_ref[...] = xs[:, :_C]
    xb_ref[...] = xs[:, _C:]
    norms_ref[...] = jnp.concatenate([ns, nd], axis=1)


_scale_split_call = pl.pallas_call(
    _scale_split_body,
    grid=(_N // _R,),
    in_specs=[
        pl.BlockSpec((_R, _F), lambda i: (i, 0)),
        pl.BlockSpec((_R, 2), lambda i: (i, 0)),
    ],
    out_specs=[
        pl.BlockSpec((_R, 2), lambda i: (i, 0)),
        pl.BlockSpec((_R, _C), lambda i: (i, 0)),
        pl.BlockSpec((_R, _C), lambda i: (i, 0)),
    ],
    out_shape=[
        jax.ShapeDtypeStruct((_N, 2), jnp.float32),
        jax.ShapeDtypeStruct((_NP, _C), jnp.float32),
        jax.ShapeDtypeStruct((_NP, _C), jnp.float32),
    ],
)


def _mlp_body(s1_ref, norms_ref, w1_ref, b1_ref, w2_ref, t2_ref):
    agg = jnp.concatenate([s1_ref[0], s1_ref[1]], axis=1)
    h = agg * norms_ref[:, 1:2]
    h = jnp.dot(h, w1_ref[...], preferred_element_type=jnp.float32) + b1_ref[...]
    h = jnp.maximum(h, 0.0)
    t2 = jnp.dot(h, w2_ref[...], preferred_element_type=jnp.float32)
    t2_ref[...] = t2 * norms_ref[:, 0:1]


_mlp_call = pl.pallas_call(
    _mlp_body,
    grid=(_N // _R,),
    in_specs=[
        pl.BlockSpec((2, _R, _C), lambda i: (0, i, 0)),
        pl.BlockSpec((_R, 2), lambda i: (i, 0)),
        pl.BlockSpec((_F, _F), lambda i: (0, 0)),
        pl.BlockSpec((1, _F), lambda i: (0, 0)),
        pl.BlockSpec((_F, _C), lambda i: (0, 0)),
    ],
    out_specs=pl.BlockSpec((_R, _C), lambda i: (i, 0)),
    out_shape=jax.ShapeDtypeStruct((_NP, _C), jnp.float32),
)


def _final_body(s2_ref, norms_ref, b2_ref, out_ref):
    agg = s2_ref[0] + s2_ref[1]
    out_ref[...] = agg * norms_ref[:, 1:2] + b2_ref[...]


_final_call = pl.pallas_call(
    _final_body,
    grid=(_N // _R,),
    in_specs=[
        pl.BlockSpec((2, _R, _C), lambda i: (0, i, 0)),
        pl.BlockSpec((_R, 2), lambda i: (i, 0)),
        pl.BlockSpec((1, _C), lambda i: (0, 0)),
    ],
    out_specs=pl.BlockSpec((_R, _C), lambda i: (i, 0)),
    out_shape=jax.ShapeDtypeStruct((_N, _C), jnp.float32),
)


def kernel(x, edge_index, W1, b1, W2, b2):
    pad = _N + (jnp.arange(_EP - _E, dtype=jnp.int32) % (_NP - _N))
    src = jnp.concatenate([edge_index[0].astype(jnp.int32), pad]).reshape(_NCHUNK, _CH)
    dst = jnp.concatenate([edge_index[1].astype(jnp.int32), pad]).reshape(_NCHUNK, _CH)
    zc = jnp.zeros((_RPT,), jnp.float32)
    zr = jnp.zeros((_CH, _C), jnp.float32)

    degs = _deg_call(src, dst, zc)
    norms, xa, xb = _scale_split_call(x, degs[:, :_N].T)
    s1 = _prop1_call(xa, xb, src, dst, zr)
    t2 = _mlp_call(s1, norms, W1, b1.reshape(1, -1), W2)
    s2 = _prop2_call(t2, t2, src, dst, zr)
    return _final_call(s2, norms, b2.reshape(1, -1))


# speedup vs baseline: 2.3036x; 1.0342x over previous
import functools

import jax
import jax.numpy as jnp
from jax import lax
from jax.experimental import pallas as pl
from jax.experimental.pallas import tpu as pltpu, tpu_sc as plsc

_N = 10000
_E = 320000
_F = 128
_C = 64
_CH = 128
_EP = 327680
_NCHUNK = _EP // _CH
_CPT = _NCHUNK // 16
_CPT2 = _NCHUNK // 32
_SENT = 10239
_NP = 10240
_RPT = 640
_R = 400
_NB = 4

_mesh = plsc.VectorSubcoreMesh(
    core_axis_name="c", subcore_axis_name="s", num_cores=2, num_subcores=16)


def _deg_body(src_hbm, dst_hbm, zc_hbm, out_hbm, idx_v, ones_v, stage_v, acc_sh, dsem):
    c = lax.axis_index("c")
    s = lax.axis_index("s")
    pltpu.sync_copy(zc_hbm.at[pl.ds(0, _RPT)], stage_v)
    pltpu.sync_copy(stage_v, acc_sh.at[pl.ds(s * _RPT, _RPT)])
    for k in range(_CH // 16):
        ones_v[pl.ds(k * 16, 16)] = jnp.ones((16,), jnp.float32)

    @pl.when(c == 0)
    def _():
        pltpu.sync_copy(src_hbm.at[pl.ds(s * _CPT, _CPT)], idx_v)

    @pl.when(c == 1)
    def _():
        pltpu.sync_copy(dst_hbm.at[pl.ds(s * _CPT, _CPT)], idx_v)

    plsc.subcore_barrier()

    def body(k, carry):
        pltpu.async_copy(ones_v, acc_sh.at[idx_v.at[k]], dsem, add=True)
        return carry

    lax.fori_loop(0, _CPT, body, 0)

    def drain(k, carry):
        pltpu.make_async_copy(ones_v, acc_sh.at[idx_v.at[k]], dsem).wait()
        return carry

    lax.fori_loop(0, _CPT, drain, 0)
    plsc.subcore_barrier()
    pltpu.sync_copy(acc_sh.at[pl.ds(s * _RPT, _RPT)], stage_v)
    pltpu.sync_copy(stage_v, out_hbm.at[c, pl.ds(s * _RPT, _RPT)])


_deg_call = pl.kernel(
    _deg_body,
    out_type=jax.ShapeDtypeStruct((2, _NP), jnp.float32),
    mesh=_mesh,
    compiler_params=pltpu.CompilerParams(use_tc_tiling_on_sc=False),
    scratch_types=[
        pltpu.VMEM((_CPT, _CH), jnp.int32),
        pltpu.VMEM((_CH,), jnp.float32),
        pltpu.VMEM((_RPT,), jnp.float32),
        pltpu.VMEM_SHARED((_NP,), jnp.float32),
        pltpu.SemaphoreType.DMA,
    ],
)


def _prop_body(edge_split, ta_hbm, tb_hbm, src_hbm, dst_hbm, zr_hbm, out_hbm,
               sidx, didx, zidx, rows, zbuf, acc_sh, gsems, ssems, zsem):
    c = lax.axis_index("c")
    s = lax.axis_index("s")
    cpt = _CPT2 if edge_split else _CPT
    chunk0 = (c * 16 + s) * cpt if edge_split else s * cpt

    iota = lax.iota(jnp.int32, 16)
    for k in range(_RPT // _CH):
        for v in range(_CH // 16):
            zidx[k, pl.ds(v * 16, 16)] = iota + (s * _RPT + k * _CH + v * 16)

    pltpu.sync_copy(zr_hbm.at[pl.ds(0, _CH)], zbuf)
    for k in range(_RPT // _CH):
        pltpu.async_copy(zbuf, acc_sh.at[zidx.at[k]], zsem)
    for k in range(_RPT // _CH):
        pltpu.make_async_copy(zbuf, acc_sh.at[zidx.at[k]], zsem).wait()

    pltpu.sync_copy(src_hbm.at[pl.ds(chunk0, cpt)], sidx.at[pl.ds(0, cpt)])
    pltpu.sync_copy(dst_hbm.at[pl.ds(chunk0, cpt)], didx.at[pl.ds(0, cpt)])
    plsc.subcore_barrier()

    def run(tbl_hbm):
        def gather(k, b):
            pltpu.async_copy(tbl_hbm.at[sidx.at[k]], rows.at[b], gsems.at[b])

        def wait_gather(k, b):
            pltpu.make_async_copy(tbl_hbm.at[sidx.at[k]], rows.at[b], gsems.at[b]).wait()

        def scatter(k, b):
            pltpu.async_copy(rows.at[b], acc_sh.at[didx.at[k]], ssems.at[b], add=True)

        def wait_scatter(k, b):
            pltpu.make_async_copy(rows.at[b], acc_sh.at[didx.at[k]], ssems.at[b]).wait()

        for b in range(_NB):
            gather(b, b)

        def body(j, carry):
            k = _NB * j
            for b in range(_NB):
                wait_gather(k + b, b)
                scatter(k + b, b)
            for b in range(_NB):
                kn = k + _NB + b

                @pl.when(kn < cpt)
                def _(b=b, kn=kn):
                    wait_scatter(kn - _NB, b)
                    gather(kn, b)
            return carry

        lax.fori_loop(0, cpt // _NB, body, 0)
        for b in range(_NB):
            wait_scatter(cpt - _NB + b, b)

    @pl.when(c == 0)
    def _():
        run(ta_hbm)

    @pl.when(c == 1)
    def _():
        run(tb_hbm)

    plsc.subcore_barrier()
    for k in range(_RPT // _CH):
        pltpu.async_copy(acc_sh.at[zidx.at[k]], rows.at[k % _NB], gsems.at[k % _NB])
        pltpu.make_async_copy(acc_sh.at[zidx.at[k]], rows.at[k % _NB], gsems.at[k % _NB]).wait()
        pltpu.sync_copy(rows.at[k % _NB],
                        out_hbm.at[c, pl.ds(s * _RPT + k * _CH, _CH)])


def _make_prop(edge_split):
    return pl.kernel(
        functools.partial(_prop_body, edge_split),
        out_type=jax.ShapeDtypeStruct((2, _NP, _C), jnp.float32),
        mesh=_mesh,
        compiler_params=pltpu.CompilerParams(use_tc_tiling_on_sc=False),
        scratch_types=[
            pltpu.VMEM((_CPT, _CH), jnp.int32),
            pltpu.VMEM((_CPT, _CH), jnp.int32),
            pltpu.VMEM((_RPT // _CH, _CH), jnp.int32),
            pltpu.VMEM((_NB, _CH, _C), jnp.float32),
            pltpu.VMEM((_CH, _C), jnp.float32),
            pltpu.VMEM_SHARED((_NP, _C), jnp.float32),
            pltpu.SemaphoreType.DMA((_NB,)),
            pltpu.SemaphoreType.DMA((_NB,)),
            pltpu.SemaphoreType.DMA,
        ],
    )


_prop1_call = _make_prop(False)
_prop2_call = _make_prop(True)


def _scale_split_body(x_ref, degt_ref, norms_ref, xa_ref, xb_ref):
    ns = lax.rsqrt(jnp.maximum(degt_ref[:, 0:1], 1.0))
    nd = lax.rsqrt(jnp.maximum(degt_ref[:, 1:2], 1.0))
    xs = x_ref[...] * ns
    xa_ref[...] = xs[:, :_C]
    xb_ref[...] = xs[:, _C:]
    norms_ref[...] = jnp.concatenate([ns, nd], axis=1)


_scale_split_call = pl.pallas_call(
    _scale_split_body,
    grid=(_N // _R,),
    in_specs=[
        pl.BlockSpec((_R, _F), lambda i: (i, 0)),
        pl.BlockSpec((_R, 2), lambda i: (i, 0)),
    ],
    out_specs=[
        pl.BlockSpec((_R, 2), lambda i: (i, 0)),
        pl.BlockSpec((_R, _C), lambda i: (i, 0)),
        pl.BlockSpec((_R, _C), lambda i: (i, 0)),
    ],
    out_shape=[
        jax.ShapeDtypeStruct((_N, 2), jnp.float32),
        jax.ShapeDtypeStruct((_NP, _C), jnp.float32),
        jax.ShapeDtypeStruct((_NP, _C), jnp.float32),
    ],
)


def _mlp_body(s1_ref, norms_ref, w1_ref, b1_ref, w2_ref, t2_ref):
    agg = jnp.concatenate([s1_ref[0], s1_ref[1]], axis=1)
    h = agg * norms_ref[:, 1:2]
    h = jnp.dot(h, w1_ref[...], preferred_element_type=jnp.float32) + b1_ref[...]
    h = jnp.maximum(h, 0.0)
    t2 = jnp.dot(h, w2_ref[...], preferred_element_type=jnp.float32)
    t2_ref[...] = t2 * norms_ref[:, 0:1]


_mlp_call = pl.pallas_call(
    _mlp_body,
    grid=(_N // _R,),
    in_specs=[
        pl.BlockSpec((2, _R, _C), lambda i: (0, i, 0)),
        pl.BlockSpec((_R, 2), lambda i: (i, 0)),
        pl.BlockSpec((_F, _F), lambda i: (0, 0)),
        pl.BlockSpec((1, _F), lambda i: (0, 0)),
        pl.BlockSpec((_F, _C), lambda i: (0, 0)),
    ],
    out_specs=pl.BlockSpec((_R, _C), lambda i: (i, 0)),
    out_shape=jax.ShapeDtypeStruct((_NP, _C), jnp.float32),
)


def _final_body(s2_ref, norms_ref, b2_ref, out_ref):
    agg = s2_ref[0] + s2_ref[1]
    out_ref[...] = agg * norms_ref[:, 1:2] + b2_ref[...]


_final_call = pl.pallas_call(
    _final_body,
    grid=(_N // _R,),
    in_specs=[
        pl.BlockSpec((2, _R, _C), lambda i: (0, i, 0)),
        pl.BlockSpec((_R, 2), lambda i: (i, 0)),
        pl.BlockSpec((1, _C), lambda i: (0, 0)),
    ],
    out_specs=pl.BlockSpec((_R, _C), lambda i: (i, 0)),
    out_shape=jax.ShapeDtypeStruct((_N, _C), jnp.float32),
)


def kernel(x, edge_index, W1, b1, W2, b2):
    pad = _N + (jnp.arange(_EP - _E, dtype=jnp.int32) % (_NP - _N))
    src = jnp.concatenate([edge_index[0].astype(jnp.int32), pad]).reshape(_NCHUNK, _CH)
    dst = jnp.concatenate([edge_index[1].astype(jnp.int32), pad]).reshape(_NCHUNK, _CH)
    zc = jnp.zeros((_RPT,), jnp.float32)
    zr = jnp.zeros((_CH, _C), jnp.float32)

    degs = _deg_call(src, dst, zc)
    norms, xa, xb = _scale_split_call(x, degs[:, :_N].T)
    s1 = _prop1_call(xa, xb, src, dst, zr)
    t2 = _mlp_call(s1, norms, W1, b1.reshape(1, -1), W2)
    s2 = _prop2_call(t2, t2, src, dst, zr)
    return _final_call(s2, norms, b2.reshape(1, -1))


# speedup vs baseline: 2.3416x; 1.0165x over previous
import functools

import jax
import jax.numpy as jnp
from jax import lax
from jax.experimental import pallas as pl
from jax.experimental.pallas import tpu as pltpu, tpu_sc as plsc

_N = 10000
_E = 320000
_F = 128
_C = 64
_CH = 128
_EP = 327680
_NCHUNK = _EP // _CH
_CPT = _NCHUNK // 16
_CPT2 = _NCHUNK // 32
_SENT = 10239
_NP = 10240
_RPT = 640
_R = 400
_NB = 4

_mesh = plsc.VectorSubcoreMesh(
    core_axis_name="c", subcore_axis_name="s", num_cores=2, num_subcores=16)


def _deg_body(src_hbm, dst_hbm, zc_hbm, out_hbm, idx_v, ones_v, stage_v, acc_sh, dsem):
    c = lax.axis_index("c")
    s = lax.axis_index("s")
    pltpu.sync_copy(zc_hbm.at[pl.ds(0, _RPT)], stage_v)
    pltpu.sync_copy(stage_v, acc_sh.at[pl.ds(s * _RPT, _RPT)])
    for k in range(_CH // 16):
        ones_v[pl.ds(k * 16, 16)] = jnp.ones((16,), jnp.float32)

    @pl.when(c == 0)
    def _():
        pltpu.sync_copy(src_hbm.at[pl.ds(s * _CPT, _CPT)], idx_v)

    @pl.when(c == 1)
    def _():
        pltpu.sync_copy(dst_hbm.at[pl.ds(s * _CPT, _CPT)], idx_v)

    plsc.subcore_barrier()

    def body(k, carry):
        pltpu.async_copy(ones_v, acc_sh.at[idx_v.at[k]], dsem, add=True)
        return carry

    lax.fori_loop(0, _CPT, body, 0)

    def drain(k, carry):
        pltpu.make_async_copy(ones_v, acc_sh.at[idx_v.at[k]], dsem).wait()
        return carry

    lax.fori_loop(0, _CPT, drain, 0)
    plsc.subcore_barrier()
    pltpu.sync_copy(acc_sh.at[pl.ds(s * _RPT, _RPT)], stage_v)
    pltpu.sync_copy(stage_v, out_hbm.at[c, pl.ds(s * _RPT, _RPT)])


_deg_call = pl.kernel(
    _deg_body,
    out_type=jax.ShapeDtypeStruct((2, _NP), jnp.float32),
    mesh=_mesh,
    compiler_params=pltpu.CompilerParams(use_tc_tiling_on_sc=False),
    scratch_types=[
        pltpu.VMEM((_CPT, _CH), jnp.int32),
        pltpu.VMEM((_CH,), jnp.float32),
        pltpu.VMEM((_RPT,), jnp.float32),
        pltpu.VMEM_SHARED((_NP,), jnp.float32),
        pltpu.SemaphoreType.DMA,
    ],
)


def _prop_body(edge_split, ta_hbm, tb_hbm, src_hbm, dst_hbm, zr_hbm, out_hbm,
               sidx, didx, zidx, rows, zbuf, acc_sh, gsems, ssems, zsem):
    c = lax.axis_index("c")
    s = lax.axis_index("s")
    cpt = _CPT2 if edge_split else _CPT
    chunk0 = (c * 16 + s) * cpt if edge_split else s * cpt

    iota = lax.iota(jnp.int32, 16)
    for k in range(_RPT // _CH):
        for v in range(_CH // 16):
            zidx[k, pl.ds(v * 16, 16)] = iota + (s * _RPT + k * _CH + v * 16)

    pltpu.sync_copy(zr_hbm.at[pl.ds(0, _CH)], zbuf)
    for k in range(_RPT // _CH):
        pltpu.async_copy(zbuf, acc_sh.at[zidx.at[k]], zsem)
    pltpu.sync_copy(src_hbm.at[pl.ds(chunk0, cpt)], sidx.at[pl.ds(0, cpt)])
    pltpu.sync_copy(dst_hbm.at[pl.ds(chunk0, cpt)], didx.at[pl.ds(0, cpt)])
    for k in range(_RPT // _CH):
        pltpu.make_async_copy(zbuf, acc_sh.at[zidx.at[k]], zsem).wait()
    plsc.subcore_barrier()

    def run(tbl_hbm):
        def gather(k, b):
            pltpu.async_copy(tbl_hbm.at[sidx.at[k]], rows.at[b], gsems.at[b])

        def wait_gather(k, b):
            pltpu.make_async_copy(tbl_hbm.at[sidx.at[k]], rows.at[b], gsems.at[b]).wait()

        def scatter(k, b):
            pltpu.async_copy(rows.at[b], acc_sh.at[didx.at[k]], ssems.at[b], add=True)

        def wait_scatter(k, b):
            pltpu.make_async_copy(rows.at[b], acc_sh.at[didx.at[k]], ssems.at[b]).wait()

        for b in range(_NB):
            gather(b, b)

        def body(j, carry):
            k = _NB * j
            for b in range(_NB):
                wait_gather(k + b, b)
                scatter(k + b, b)
            for b in range(_NB):
                kn = k + _NB + b

                @pl.when(kn < cpt)
                def _(b=b, kn=kn):
                    wait_scatter(kn - _NB, b)
                    gather(kn, b)
            return carry

        lax.fori_loop(0, cpt // _NB, body, 0)
        for b in range(_NB):
            wait_scatter(cpt - _NB + b, b)

    @pl.when(c == 0)
    def _():
        run(ta_hbm)

    @pl.when(c == 1)
    def _():
        run(tb_hbm)

    plsc.subcore_barrier()
    for k in range(_NB):
        pltpu.async_copy(acc_sh.at[zidx.at[k]], rows.at[k], gsems.at[k])
    for k in range(_RPT // _CH):
        b = k % _NB
        pltpu.make_async_copy(acc_sh.at[zidx.at[k]], rows.at[b], gsems.at[b]).wait()
        pltpu.async_copy(rows.at[b],
                         out_hbm.at[c, pl.ds(s * _RPT + k * _CH, _CH)], ssems.at[b])
        if k + _NB < _RPT // _CH:
            pltpu.make_async_copy(
                rows.at[b], out_hbm.at[c, pl.ds(s * _RPT + k * _CH, _CH)],
                ssems.at[b]).wait()
            pltpu.async_copy(acc_sh.at[zidx.at[k + _NB]], rows.at[b], gsems.at[b])
    for k in range(_RPT // _CH):
        b = k % _NB
        if k + _NB >= _RPT // _CH:
            pltpu.make_async_copy(
                rows.at[b], out_hbm.at[c, pl.ds(s * _RPT + k * _CH, _CH)],
                ssems.at[b]).wait()


def _make_prop(edge_split):
    return pl.kernel(
        functools.partial(_prop_body, edge_split),
        out_type=jax.ShapeDtypeStruct((2, _NP, _C), jnp.float32),
        mesh=_mesh,
        compiler_params=pltpu.CompilerParams(use_tc_tiling_on_sc=False),
        scratch_types=[
            pltpu.VMEM((_CPT, _CH), jnp.int32),
            pltpu.VMEM((_CPT, _CH), jnp.int32),
            pltpu.VMEM((_RPT // _CH, _CH), jnp.int32),
            pltpu.VMEM((_NB, _CH, _C), jnp.float32),
            pltpu.VMEM((_CH, _C), jnp.float32),
            pltpu.VMEM_SHARED((_NP, _C), jnp.float32),
            pltpu.SemaphoreType.DMA((_NB,)),
            pltpu.SemaphoreType.DMA((_NB,)),
            pltpu.SemaphoreType.DMA,
        ],
    )


_prop1_call = _make_prop(False)
_prop2_call = _make_prop(True)


def _scale_split_body(x_ref, degt_ref, norms_ref, xa_ref, xb_ref):
    ns = lax.rsqrt(jnp.maximum(degt_ref[:, 0:1], 1.0))
    nd = lax.rsqrt(jnp.maximum(degt_ref[:, 1:2], 1.0))
    xs = x_ref[...] * ns
    xa_ref[...] = xs[:, :_C]
    xb_ref[...] = xs[:, _C:]
    norms_ref[...] = jnp.concatenate([ns, nd], axis=1)


_scale_split_call = pl.pallas_call(
    _scale_split_body,
    grid=(_N // _R,),
    in_specs=[
        pl.BlockSpec((_R, _F), lambda i: (i, 0)),
        pl.BlockSpec((_R, 2), lambda i: (i, 0)),
    ],
    out_specs=[
        pl.BlockSpec((_R, 2), lambda i: (i, 0)),
        pl.BlockSpec((_R, _C), lambda i: (i, 0)),
        pl.BlockSpec((_R, _C), lambda i: (i, 0)),
    ],
    out_shape=[
        jax.ShapeDtypeStruct((_N, 2), jnp.float32),
        jax.ShapeDtypeStruct((_NP, _C), jnp.float32),
        jax.ShapeDtypeStruct((_NP, _C), jnp.float32),
    ],
)


def _mlp_body(s1_ref, norms_ref, w1_ref, b1_ref, w2_ref, t2_ref):
    agg = jnp.concatenate([s1_ref[0], s1_ref[1]], axis=1)
    h = agg * norms_ref[:, 1:2]
    h = jnp.dot(h, w1_ref[...], preferred_element_type=jnp.float32) + b1_ref[...]
    h = jnp.maximum(h, 0.0)
    t2 = jnp.dot(h, w2_ref[...], preferred_element_type=jnp.float32)
    t2_ref[...] = t2 * norms_ref[:, 0:1]


_mlp_call = pl.pallas_call(
    _mlp_body,
    grid=(_N // _R,),
    in_specs=[
        pl.BlockSpec((2, _R, _C), lambda i: (0, i, 0)),
        pl.BlockSpec((_R, 2), lambda i: (i, 0)),
        pl.BlockSpec((_F, _F), lambda i: (0, 0)),
        pl.BlockSpec((1, _F), lambda i: (0, 0)),
        pl.BlockSpec((_F, _C), lambda i: (0, 0)),
    ],
    out_specs=pl.BlockSpec((_R, _C), lambda i: (i, 0)),
    out_shape=jax.ShapeDtypeStruct((_NP, _C), jnp.float32),
)


def _final_body(s2_ref, norms_ref, b2_ref, out_ref):
    agg = s2_ref[0] + s2_ref[1]
    out_ref[...] = agg * norms_ref[:, 1:2] + b2_ref[...]


_final_call = pl.pallas_call(
    _final_body,
    grid=(_N // _R,),
    in_specs=[
        pl.BlockSpec((2, _R, _C), lambda i: (0, i, 0)),
        pl.BlockSpec((_R, 2), lambda i: (i, 0)),
        pl.BlockSpec((1, _C), lambda i: (0, 0)),
    ],
    out_specs=pl.BlockSpec((_R, _C), lambda i: (i, 0)),
    out_shape=jax.ShapeDtypeStruct((_N, _C), jnp.float32),
)


def kernel(x, edge_index, W1, b1, W2, b2):
    pad = _N + (jnp.arange(_EP - _E, dtype=jnp.int32) % (_NP - _N))
    src = jnp.concatenate([edge_index[0].astype(jnp.int32), pad]).reshape(_NCHUNK, _CH)
    dst = jnp.concatenate([edge_index[1].astype(jnp.int32), pad]).reshape(_NCHUNK, _CH)
    zc = jnp.zeros((_RPT,), jnp.float32)
    zr = jnp.zeros((_CH, _C), jnp.float32)

    degs = _deg_call(src, dst, zc)
    norms, xa, xb = _scale_split_call(x, degs[:, :_N].T)
    s1 = _prop1_call(xa, xb, src, dst, zr)
    t2 = _mlp_call(s1, norms, W1, b1.reshape(1, -1), W2)
    s2 = _prop2_call(t2, t2, src, dst, zr)
    return _final_call(s2, norms, b2.reshape(1, -1))
